# baseline (speedup 1.0000x reference)
import functools

import jax
import jax.numpy as jnp
from jax.experimental import pallas as pl
from jax.experimental.pallas import tpu as pltpu

N = 1024
VEC = 256
OUT = 64
EDGE = 4
IB = 8
R = IB * N
GRID = N // IB


def _body(x_ref, adj_ref, adjt_ref, wni_ref, wno_ref,
          wei_ref, weo_ref, wagg_ref, bias_ref, sela_ref,
          fin_hbm, fout_hbm,
          out_ref, min_hbm, mout_hbm,
          finb, foutb, minb, moutb,
          sin_s, sout_s, nin_s, nout_s, eins_s, eouta_s,
          sem_fin, sem_fout, sem_min, sem_mout):
    i = pl.program_id(0)
    slot = jax.lax.rem(i, 2)
    rows = pl.ds(i * IB, IB)

    fin2 = fin_hbm.reshape(N * N, EDGE)
    fout2 = fout_hbm.reshape(N * N, EDGE)
    min2 = min_hbm.reshape(N * N, OUT)
    mout2 = mout_hbm.reshape(N * N, OUT)

    def in_cp(src2, buf, sem, blk, s):
        return pltpu.make_async_copy(
            src2.at[pl.ds(blk * R, R)], buf.at[s], sem.at[s])

    def out_cp(buf, dst2, sem, blk, s):
        return pltpu.make_async_copy(
            buf.at[s], dst2.at[pl.ds(blk * R, R)], sem.at[s])

    @pl.when(i == 0)
    def _prologue():
        in_cp(fin2, finb, sem_fin, 0, 0).start()
        in_cp(fout2, foutb, sem_fout, 0, 0).start()
        in_cp(fin2, finb, sem_fin, 1, 1).start()
        in_cp(fout2, foutb, sem_fout, 1, 1).start()
        xv = x_ref[...]
        sin_s[...] = jnp.dot(xv, wni_ref[...], preferred_element_type=jnp.float32)
        sout_s[...] = jnp.dot(xv, wno_ref[...], preferred_element_type=jnp.float32)
        eouta_s[...] = jnp.zeros_like(eouta_s)

    in_cp(fin2, finb, sem_fin, i, slot).wait()
    in_cp(fout2, foutb, sem_fout, i, slot).wait()

    @pl.when(i >= 2)
    def _drain_prev():
        out_cp(minb, min2, sem_min, i - 2, slot).wait()
        out_cp(moutb, mout2, sem_mout, i - 2, slot).wait()

    wei = wei_ref[...]
    weo = weo_ref[...]

    fin = finb[slot]
    minb[slot] = jnp.dot(fin, wei, preferred_element_type=jnp.float32)
    fin_i = jnp.dot(sela_ref[...], fin, preferred_element_type=jnp.float32)
    eins_s[rows, :] = jnp.dot(fin_i, wei, preferred_element_type=jnp.float32)

    fout = foutb[slot]
    moutb[slot] = jnp.dot(fout, weo, preferred_element_type=jnp.float32)
    eouta_s[...] += fout.reshape(IB, N, EDGE).sum(axis=0)

    nout_s[rows, :] = jnp.dot(adj_ref[...], sout_s[...],
                              preferred_element_type=jnp.float32)
    nin_s[rows, :] = jnp.dot(adjt_ref[...], sin_s[...],
                             preferred_element_type=jnp.float32)

    out_cp(minb, min2, sem_min, i, slot).start()
    out_cp(moutb, mout2, sem_mout, i, slot).start()

    @pl.when(i + 2 < GRID)
    def _prefetch():
        in_cp(fin2, finb, sem_fin, i + 2, slot).start()
        in_cp(fout2, foutb, sem_fout, i + 2, slot).start()

    @pl.when(i == GRID - 1)
    def _finish():
        out_cp(minb, min2, sem_min, GRID - 2, 1 - slot).wait()
        out_cp(moutb, mout2, sem_mout, GRID - 2, 1 - slot).wait()
        out_cp(minb, min2, sem_min, GRID - 1, slot).wait()
        out_cp(moutb, mout2, sem_mout, GRID - 1, slot).wait()
        eout = jnp.dot(eouta_s[...], weo, preferred_element_type=jnp.float32)
        wagg = wagg_ref[...]
        h = OUT // 2
        acc = jnp.dot(nin_s[...], wagg[0:h, :],
                      preferred_element_type=jnp.float32)
        acc += jnp.dot(nout_s[...], wagg[h:2 * h, :],
                       preferred_element_type=jnp.float32)
        acc += jnp.dot(eins_s[...], wagg[2 * h:2 * h + OUT, :],
                       preferred_element_type=jnp.float32)
        acc += jnp.dot(eout, wagg[2 * h + OUT:, :],
                       preferred_element_type=jnp.float32)
        out_ref[...] = jnp.maximum(acc + bias_ref[...], 0.0)


@jax.jit
def kernel(x, adj_matrix, edge_in_feat_matrix, edge_out_feat_matrix,
           weight_node_in, weight_node_out, weight_edge_in, weight_edge_out,
           weight_aggressive, bias):
    adj_t = adj_matrix.T
    bias2 = bias.reshape(1, OUT)
    sela = jnp.kron(jnp.eye(IB, dtype=jnp.float32),
                    jnp.ones((1, N), jnp.float32))

    in_specs = [
        pl.BlockSpec((N, VEC), lambda i: (0, 0)),
        pl.BlockSpec((IB, N), lambda i: (i, 0)),
        pl.BlockSpec((IB, N), lambda i: (i, 0)),
        pl.BlockSpec((VEC, OUT // 2), lambda i: (0, 0)),
        pl.BlockSpec((VEC, OUT // 2), lambda i: (0, 0)),
        pl.BlockSpec((EDGE, OUT), lambda i: (0, 0)),
        pl.BlockSpec((EDGE, OUT), lambda i: (0, 0)),
        pl.BlockSpec((3 * OUT, OUT), lambda i: (0, 0)),
        pl.BlockSpec((1, OUT), lambda i: (0, 0)),
        pl.BlockSpec((IB, R), lambda i: (0, 0)),
        pl.BlockSpec(memory_space=pl.ANY),
        pl.BlockSpec(memory_space=pl.ANY),
    ]
    out_specs = [
        pl.BlockSpec((N, OUT), lambda i: (0, 0)),
        pl.BlockSpec(memory_space=pl.ANY),
        pl.BlockSpec(memory_space=pl.ANY),
    ]

    out, min3, mout3 = pl.pallas_call(
        _body,
        grid=(GRID,),
        in_specs=in_specs,
        out_specs=out_specs,
        out_shape=[
            jax.ShapeDtypeStruct((N, OUT), jnp.float32),
            jax.ShapeDtypeStruct((N, N, OUT), jnp.float32),
            jax.ShapeDtypeStruct((N, N, OUT), jnp.float32),
        ],
        scratch_shapes=[
            pltpu.VMEM((2, R, EDGE), jnp.float32),
            pltpu.VMEM((2, R, EDGE), jnp.float32),
            pltpu.VMEM((2, R, OUT), jnp.float32),
            pltpu.VMEM((2, R, OUT), jnp.float32),
            pltpu.VMEM((N, OUT // 2), jnp.float32),
            pltpu.VMEM((N, OUT // 2), jnp.float32),
            pltpu.VMEM((N, OUT // 2), jnp.float32),
            pltpu.VMEM((N, OUT // 2), jnp.float32),
            pltpu.VMEM((N, OUT), jnp.float32),
            pltpu.VMEM((N, EDGE), jnp.float32),
            pltpu.SemaphoreType.DMA((2,)),
            pltpu.SemaphoreType.DMA((2,)),
            pltpu.SemaphoreType.DMA((2,)),
            pltpu.SemaphoreType.DMA((2,)),
        ],
        compiler_params=pltpu.CompilerParams(
            dimension_semantics=("arbitrary",),
        ),
    )(x, adj_matrix, adj_t,
      weight_node_in, weight_node_out, weight_edge_in, weight_edge_out,
      weight_aggressive, bias2, sela,
      edge_in_feat_matrix, edge_out_feat_matrix)

    return out, min3, mout3


# speedup vs baseline: 8.8927x; 8.8927x over previous
import functools

import jax
import jax.numpy as jnp
from jax.experimental import pallas as pl
from jax.experimental.pallas import tpu as pltpu

N = 1024
VEC = 256
OUT = 64
EDGE = 4
IB = 8
GRID = N // IB

_NT = (((1,), (1,)), ((), ()))


def _body(x_ref, adj_ref, fin_ref, fout_ref, wnit_ref, wnot_ref,
          weit_ref, weot_ref, waggt_ref, bias_ref,
          out_ref, min_ref, mout_ref,
          sint_s, soutt_s, eins_s, eouta_s):
    i = pl.program_id(0)
    rows = pl.ds(i * IB, IB)

    @pl.when(i == 0)
    def _init():
        xv = x_ref[...]
        sint_s[...] = jax.lax.dot_general(
            wnit_ref[...], xv, _NT, preferred_element_type=jnp.float32)
        soutt_s[...] = jax.lax.dot_general(
            wnot_ref[...], xv, _NT, preferred_element_type=jnp.float32)
        eouta_s[...] = jnp.zeros_like(eouta_s)

    weit = weit_ref[...]
    weot = weot_ref[...]

    fin = fin_ref[...]
    for k in range(IB):
        min_ref[k] = jnp.dot(weit, fin[k], preferred_element_type=jnp.float32)
    eins_s[rows, :] = fin.sum(axis=2)

    fout = fout_ref[...]
    for k in range(IB):
        mout_ref[k] = jnp.dot(weot, fout[k], preferred_element_type=jnp.float32)
    eouta_s[...] += fout.sum(axis=0)

    @pl.when(i == GRID - 1)
    def _finish():
        adj = adj_ref[...]
        nin_t = jnp.dot(sint_s[...], adj, preferred_element_type=jnp.float32)
        nout_t = jax.lax.dot_general(
            soutt_s[...], adj, _NT, preferred_element_type=jnp.float32)
        eins_t = jax.lax.dot_general(
            weit, eins_s[...], _NT, preferred_element_type=jnp.float32)
        eout_t = jnp.dot(weot, eouta_s[...], preferred_element_type=jnp.float32)
        waggt = waggt_ref[...]
        h = OUT // 2
        acc = jnp.dot(waggt[:, 0:h], nin_t, preferred_element_type=jnp.float32)
        acc += jnp.dot(waggt[:, h:2 * h], nout_t,
                       preferred_element_type=jnp.float32)
        acc += jnp.dot(waggt[:, 2 * h:2 * h + OUT], eins_t,
                       preferred_element_type=jnp.float32)
        acc += jnp.dot(waggt[:, 2 * h + OUT:], eout_t,
                       preferred_element_type=jnp.float32)
        out_ref[...] = jnp.maximum(acc + bias_ref[...], 0.0)


@jax.jit
def kernel(x, adj_matrix, edge_in_feat_matrix, edge_out_feat_matrix,
           weight_node_in, weight_node_out, weight_edge_in, weight_edge_out,
           weight_aggressive, bias):
    fin_t = jnp.transpose(edge_in_feat_matrix, (0, 2, 1))
    fout_t = jnp.transpose(edge_out_feat_matrix, (0, 2, 1))
    wni_t = weight_node_in.T
    wno_t = weight_node_out.T
    wei_t = weight_edge_in.T
    weo_t = weight_edge_out.T
    wagg_t = weight_aggressive.T
    bias_c = bias.reshape(OUT, 1)

    in_specs = [
        pl.BlockSpec((N, VEC), lambda i: (0, 0)),
        pl.BlockSpec((N, N), lambda i: (0, 0)),
        pl.BlockSpec((IB, EDGE, N), lambda i: (i, 0, 0)),
        pl.BlockSpec((IB, EDGE, N), lambda i: (i, 0, 0)),
        pl.BlockSpec((OUT // 2, VEC), lambda i: (0, 0)),
        pl.BlockSpec((OUT // 2, VEC), lambda i: (0, 0)),
        pl.BlockSpec((OUT, EDGE), lambda i: (0, 0)),
        pl.BlockSpec((OUT, EDGE), lambda i: (0, 0)),
        pl.BlockSpec((OUT, 3 * OUT), lambda i: (0, 0)),
        pl.BlockSpec((OUT, 1), lambda i: (0, 0)),
    ]
    out_specs = [
        pl.BlockSpec((OUT, N), lambda i: (0, 0)),
        pl.BlockSpec((IB, OUT, N), lambda i: (i, 0, 0)),
        pl.BlockSpec((IB, OUT, N), lambda i: (i, 0, 0)),
    ]

    out_t, min_t, mout_t = pl.pallas_call(
        _body,
        grid=(GRID,),
        in_specs=in_specs,
        out_specs=out_specs,
        out_shape=[
            jax.ShapeDtypeStruct((OUT, N), jnp.float32),
            jax.ShapeDtypeStruct((N, OUT, N), jnp.float32),
            jax.ShapeDtypeStruct((N, OUT, N), jnp.float32),
        ],
        scratch_shapes=[
            pltpu.VMEM((OUT // 2, N), jnp.float32),
            pltpu.VMEM((OUT // 2, N), jnp.float32),
            pltpu.VMEM((N, EDGE), jnp.float32),
            pltpu.VMEM((EDGE, N), jnp.float32),
        ],
        compiler_params=pltpu.CompilerParams(
            dimension_semantics=("arbitrary",),
        ),
    )(x, adj_matrix, fin_t, fout_t, wni_t, wno_t, wei_t, weo_t,
      wagg_t, bias_c)

    return (out_t.T,
            jnp.transpose(min_t, (0, 2, 1)),
            jnp.transpose(mout_t, (0, 2, 1)))


# speedup vs baseline: 10.2906x; 1.1572x over previous
import functools

import jax
import jax.numpy as jnp
from jax.experimental import pallas as pl
from jax.experimental.pallas import tpu as pltpu

N = 1024
VEC = 256
OUT = 64
EDGE = 4
IB = 16
GRID = N // IB

_NT = (((1,), (1,)), ((), ()))


def _body(x_ref, adj_ref, fin_ref, fout_ref, wnit_ref, wnot_ref,
          weit_ref, weot_ref, waggt_ref, bias_ref,
          out_ref, min_ref, mout_ref,
          sint_s, soutt_s, eins_s, eouta_s):
    i = pl.program_id(0)
    rows = pl.ds(i * IB, IB)

    @pl.when(i == 0)
    def _init():
        xv = x_ref[...]
        sint_s[...] = jax.lax.dot_general(
            wnit_ref[...], xv, _NT, preferred_element_type=jnp.float32)
        soutt_s[...] = jax.lax.dot_general(
            wnot_ref[...], xv, _NT, preferred_element_type=jnp.float32)
        eouta_s[...] = jnp.zeros_like(eouta_s)

    weit = weit_ref[...]
    weot = weot_ref[...]

    fin = fin_ref[...]
    for k in range(IB):
        min_ref[k] = jnp.dot(weit, fin[k], preferred_element_type=jnp.float32)
    eins_s[rows, :] = fin.sum(axis=2)

    fout = fout_ref[...]
    for k in range(IB):
        mout_ref[k] = jnp.dot(weot, fout[k], preferred_element_type=jnp.float32)
    eouta_s[...] += fout.sum(axis=0)

    @pl.when(i == GRID - 1)
    def _finish():
        adj = adj_ref[...]
        nin_t = jnp.dot(sint_s[...], adj, preferred_element_type=jnp.float32)
        nout_t = jax.lax.dot_general(
            soutt_s[...], adj, _NT, preferred_element_type=jnp.float32)
        eins_t = jax.lax.dot_general(
            weit, eins_s[...], _NT, preferred_element_type=jnp.float32)
        eout_t = jnp.dot(weot, eouta_s[...], preferred_element_type=jnp.float32)
        waggt = waggt_ref[...]
        h = OUT // 2
        acc = jnp.dot(waggt[:, 0:h], nin_t, preferred_element_type=jnp.float32)
        acc += jnp.dot(waggt[:, h:2 * h], nout_t,
                       preferred_element_type=jnp.float32)
        acc += jnp.dot(waggt[:, 2 * h:2 * h + OUT], eins_t,
                       preferred_element_type=jnp.float32)
        acc += jnp.dot(waggt[:, 2 * h + OUT:], eout_t,
                       preferred_element_type=jnp.float32)
        out_ref[...] = jnp.maximum(acc + bias_ref[...], 0.0)


@jax.jit
def kernel(x, adj_matrix, edge_in_feat_matrix, edge_out_feat_matrix,
           weight_node_in, weight_node_out, weight_edge_in, weight_edge_out,
           weight_aggressive, bias):
    fin_t = jnp.transpose(edge_in_feat_matrix, (0, 2, 1))
    fout_t = jnp.transpose(edge_out_feat_matrix, (0, 2, 1))
    wni_t = weight_node_in.T
    wno_t = weight_node_out.T
    wei_t = weight_edge_in.T
    weo_t = weight_edge_out.T
    wagg_t = weight_aggressive.T
    bias_c = bias.reshape(OUT, 1)

    in_specs = [
        pl.BlockSpec((N, VEC), lambda i: (0, 0)),
        pl.BlockSpec((N, N), lambda i: (0, 0)),
        pl.BlockSpec((IB, EDGE, N), lambda i: (i, 0, 0)),
        pl.BlockSpec((IB, EDGE, N), lambda i: (i, 0, 0)),
        pl.BlockSpec((OUT // 2, VEC), lambda i: (0, 0)),
        pl.BlockSpec((OUT // 2, VEC), lambda i: (0, 0)),
        pl.BlockSpec((OUT, EDGE), lambda i: (0, 0)),
        pl.BlockSpec((OUT, EDGE), lambda i: (0, 0)),
        pl.BlockSpec((OUT, 3 * OUT), lambda i: (0, 0)),
        pl.BlockSpec((OUT, 1), lambda i: (0, 0)),
    ]
    out_specs = [
        pl.BlockSpec((OUT, N), lambda i: (0, 0)),
        pl.BlockSpec((IB, OUT, N), lambda i: (i, 0, 0)),
        pl.BlockSpec((IB, OUT, N), lambda i: (i, 0, 0)),
    ]

    out_t, min_t, mout_t = pl.pallas_call(
        _body,
        grid=(GRID,),
        in_specs=in_specs,
        out_specs=out_specs,
        out_shape=[
            jax.ShapeDtypeStruct((OUT, N), jnp.float32),
            jax.ShapeDtypeStruct((N, OUT, N), jnp.float32),
            jax.ShapeDtypeStruct((N, OUT, N), jnp.float32),
        ],
        scratch_shapes=[
            pltpu.VMEM((OUT // 2, N), jnp.float32),
            pltpu.VMEM((OUT // 2, N), jnp.float32),
            pltpu.VMEM((N, EDGE), jnp.float32),
            pltpu.VMEM((EDGE, N), jnp.float32),
        ],
        compiler_params=pltpu.CompilerParams(
            dimension_semantics=("arbitrary",),
        ),
    )(x, adj_matrix, fin_t, fout_t, wni_t, wno_t, wei_t, weo_t,
      wagg_t, bias_c)

    return (out_t.T,
            jnp.transpose(min_t, (0, 2, 1)),
            jnp.transpose(mout_t, (0, 2, 1)))


# speedup vs baseline: 10.5327x; 1.0235x over previous
import functools

import jax
import jax.numpy as jnp
from jax.experimental import pallas as pl
from jax.experimental.pallas import tpu as pltpu

N = 1024
VEC = 256
OUT = 64
EDGE = 4
IB = 32
GRID = N // IB

_NT = (((1,), (1,)), ((), ()))


def _body(x_ref, adj_ref, fin_ref, fout_ref, wnit_ref, wnot_ref,
          weit_ref, weot_ref, waggt_ref, bias_ref,
          out_ref, min_ref, mout_ref,
          sint_s, soutt_s, eins_s, eouta_s):
    i = pl.program_id(0)
    rows = pl.ds(i * IB, IB)

    @pl.when(i == 0)
    def _init():
        xv = x_ref[...]
        sint_s[...] = jax.lax.dot_general(
            wnit_ref[...], xv, _NT, preferred_element_type=jnp.float32)
        soutt_s[...] = jax.lax.dot_general(
            wnot_ref[...], xv, _NT, preferred_element_type=jnp.float32)
        eouta_s[...] = jnp.zeros_like(eouta_s)

    weit = weit_ref[...]
    weot = weot_ref[...]

    fin = fin_ref[...]
    for k in range(IB):
        min_ref[k] = jnp.dot(weit, fin[k], preferred_element_type=jnp.float32)
    eins_s[rows, :] = fin.sum(axis=2)

    fout = fout_ref[...]
    for k in range(IB):
        mout_ref[k] = jnp.dot(weot, fout[k], preferred_element_type=jnp.float32)
    eouta_s[...] += fout.sum(axis=0)

    @pl.when(i == GRID - 1)
    def _finish():
        adj = adj_ref[...]
        nin_t = jnp.dot(sint_s[...], adj, preferred_element_type=jnp.float32)
        nout_t = jax.lax.dot_general(
            soutt_s[...], adj, _NT, preferred_element_type=jnp.float32)
        eins_t = jax.lax.dot_general(
            weit, eins_s[...], _NT, preferred_element_type=jnp.float32)
        eout_t = jnp.dot(weot, eouta_s[...], preferred_element_type=jnp.float32)
        waggt = waggt_ref[...]
        h = OUT // 2
        acc = jnp.dot(waggt[:, 0:h], nin_t, preferred_element_type=jnp.float32)
        acc += jnp.dot(waggt[:, h:2 * h], nout_t,
                       preferred_element_type=jnp.float32)
        acc += jnp.dot(waggt[:, 2 * h:2 * h + OUT], eins_t,
                       preferred_element_type=jnp.float32)
        acc += jnp.dot(waggt[:, 2 * h + OUT:], eout_t,
                       preferred_element_type=jnp.float32)
        out_ref[...] = jnp.maximum(acc + bias_ref[...], 0.0)


@jax.jit
def kernel(x, adj_matrix, edge_in_feat_matrix, edge_out_feat_matrix,
           weight_node_in, weight_node_out, weight_edge_in, weight_edge_out,
           weight_aggressive, bias):
    fin_t = jnp.transpose(edge_in_feat_matrix, (0, 2, 1))
    fout_t = jnp.transpose(edge_out_feat_matrix, (0, 2, 1))
    wni_t = weight_node_in.T
    wno_t = weight_node_out.T
    wei_t = weight_edge_in.T
    weo_t = weight_edge_out.T
    wagg_t = weight_aggressive.T
    bias_c = bias.reshape(OUT, 1)

    in_specs = [
        pl.BlockSpec((N, VEC), lambda i: (0, 0)),
        pl.BlockSpec((N, N), lambda i: (0, 0)),
        pl.BlockSpec((IB, EDGE, N), lambda i: (i, 0, 0)),
        pl.BlockSpec((IB, EDGE, N), lambda i: (i, 0, 0)),
        pl.BlockSpec((OUT // 2, VEC), lambda i: (0, 0)),
        pl.BlockSpec((OUT // 2, VEC), lambda i: (0, 0)),
        pl.BlockSpec((OUT, EDGE), lambda i: (0, 0)),
        pl.BlockSpec((OUT, EDGE), lambda i: (0, 0)),
        pl.BlockSpec((OUT, 3 * OUT), lambda i: (0, 0)),
        pl.BlockSpec((OUT, 1), lambda i: (0, 0)),
    ]
    out_specs = [
        pl.BlockSpec((OUT, N), lambda i: (0, 0)),
        pl.BlockSpec((IB, OUT, N), lambda i: (i, 0, 0)),
        pl.BlockSpec((IB, OUT, N), lambda i: (i, 0, 0)),
    ]

    out_t, min_t, mout_t = pl.pallas_call(
        _body,
        grid=(GRID,),
        in_specs=in_specs,
        out_specs=out_specs,
        out_shape=[
            jax.ShapeDtypeStruct((OUT, N), jnp.float32),
            jax.ShapeDtypeStruct((N, OUT, N), jnp.float32),
            jax.ShapeDtypeStruct((N, OUT, N), jnp.float32),
        ],
        scratch_shapes=[
            pltpu.VMEM((OUT // 2, N), jnp.float32),
            pltpu.VMEM((OUT // 2, N), jnp.float32),
            pltpu.VMEM((N, EDGE), jnp.float32),
            pltpu.VMEM((EDGE, N), jnp.float32),
        ],
        compiler_params=pltpu.CompilerParams(
            dimension_semantics=("arbitrary",),
        ),
    )(x, adj_matrix, fin_t, fout_t, wni_t, wno_t, wei_t, weo_t,
      wagg_t, bias_c)

    return (out_t.T,
            jnp.transpose(min_t, (0, 2, 1)),
            jnp.transpose(mout_t, (0, 2, 1)))
